# baseline (speedup 1.0000x reference)
import functools

import jax
import jax.numpy as jnp
from jax import lax
from jax.experimental import pallas as pl
from jax.experimental.pallas import tpu as pltpu
from jax.experimental.pallas import tpu_sc as plsc

_K = 32
_EPS = 1e-5
_ROWS = 2048


def _fps_body(npoint, b, n, xyz_ref, iota_ref, idx_ref):
    x = xyz_ref[0]
    y = xyz_ref[1]
    z = xyz_ref[2]
    iota = iota_ref[...]
    eye = (lax.broadcasted_iota(jnp.int32, (b, b), 0)
           == lax.broadcasted_iota(jnp.int32, (b, b), 1))

    def _red(a, op):
        return op(op(a, axis=2, keepdims=True), axis=1, keepdims=True)

    def body(i, carry):
        dist, far = carry
        far_row = jnp.sum(jnp.where(eye, jnp.broadcast_to(far[:, :, 0], (b, b)), 0),
                          axis=0, keepdims=True)
        idx_ref[pl.ds(i, 1), :] = far_row
        oh = iota == far
        cx = _red(jnp.where(oh, x, 0.0), jnp.sum)
        cy = _red(jnp.where(oh, y, 0.0), jnp.sum)
        cz = _red(jnp.where(oh, z, 0.0), jnp.sum)
        d = (x - cx) ** 2 + (y - cy) ** 2 + (z - cz) ** 2
        dist = jnp.minimum(dist, d)
        m = _red(dist, jnp.max)
        far = _red(jnp.where(dist == m, iota, n), jnp.min)
        return dist, far

    lax.fori_loop(0, npoint, body,
                  (jnp.full((b, 8, n // 8), 1e10, jnp.float32),
                   jnp.zeros((b, 1, 1), jnp.int32)))


def _fps(xyz3, npoint):
    _, b, n = xyz3.shape
    xyz4 = xyz3.reshape(3, b, 8, n // 8)
    iota = jnp.arange(n, dtype=jnp.int32).reshape(8, n // 8)
    out = pl.pallas_call(
        functools.partial(_fps_body, npoint, b, n),
        out_shape=jax.ShapeDtypeStruct((npoint, b), jnp.int32),
    )(xyz4, iota)
    return out.T


def _knn_body(n, qb, x_ref, q_ref, o_ref):
    px = x_ref[0]
    sq = jnp.sum(px * px, axis=0, keepdims=True)
    q = q_ref[0]
    d = sq - 2.0 * jnp.dot(q, px, preferred_element_type=jnp.float32)
    iota = lax.broadcasted_iota(jnp.int32, (qb, n), 1)

    sels = []
    for _ in range(_K):
        m = jnp.min(d, axis=1, keepdims=True)
        sel = jnp.min(jnp.where(d == m, iota, n), axis=1, keepdims=True)
        sels.append(sel)
        d = jnp.where(iota == sel, jnp.inf, d)
    o_ref[0] = jnp.concatenate(sels, axis=1)


def _knn(points3, queries):
    b, _, n = points3.shape
    s = queries.shape[1]
    qb = 512
    return pl.pallas_call(
        functools.partial(_knn_body, n, qb),
        grid=(b, s // qb),
        in_specs=[
            pl.BlockSpec((1, 3, n), lambda i, j: (i, 0, 0)),
            pl.BlockSpec((1, qb, 3), lambda i, j: (i, j, 0)),
        ],
        out_specs=pl.BlockSpec((1, qb, _K), lambda i, j: (i, j, 0)),
        out_shape=jax.ShapeDtypeStruct((b, s, _K), jnp.int32),
    )(points3, queries)


def _sc_gather(table, idx):
    v, d = table.shape
    bt = idx.shape[0]
    info = plsc.get_sparse_core_info()
    nw = info.num_cores * info.num_subcores
    b_per_w = bt // nw
    cb = b_per_w
    while cb * d * 4 > 262144:
        cb //= 2
    nchunks = b_per_w // cb
    mesh = plsc.VectorSubcoreMesh(core_axis_name="c", subcore_axis_name="s")

    @functools.partial(
        pl.kernel, mesh=mesh,
        out_type=jax.ShapeDtypeStruct((bt, d), jnp.float32),
        compiler_params=pltpu.CompilerParams(use_tc_tiling_on_sc=False),
        scratch_types=[
            pltpu.VMEM((cb,), jnp.int32),
            pltpu.VMEM((cb, d), jnp.float32),
            pltpu.SemaphoreType.DMA,
        ],
    )
    def k(table_hbm, idx_hbm, out_hbm, idx_v, rows_v, sem):
        wid = lax.axis_index("s") * info.num_cores + lax.axis_index("c")
        base = wid * b_per_w

        def body(c, carry):
            off = base + c * cb
            pltpu.sync_copy(idx_hbm.at[pl.ds(off, cb)], idx_v)
            pltpu.async_copy(table_hbm.at[idx_v], rows_v, sem).wait()
            pltpu.sync_copy(rows_v, out_hbm.at[pl.ds(off, cb)])
            return carry

        lax.fori_loop(0, nchunks, body, 0)

    return k(table, idx)


def _mlp_body(rows, dch, o, g_ref, q_ref, w0_ref, b0_ref, w1_ref, b1_ref,
              w2_ref, b2_ref, o_ref, z_ref):
    nch = rows // _ROWS
    rq = _ROWS // _K
    ninv = 1.0 / rows

    def phase0(c, carry):
        s1, s2 = carry
        base = pl.multiple_of(c * _ROWS, _ROWS)
        x0 = g_ref[0, :, pl.ds(base, _ROWS)] - q_ref[0, :, pl.ds(base, _ROWS)]
        z = jnp.dot(w0_ref[...], x0, preferred_element_type=jnp.float32) + b0_ref[...]
        z_ref[:, pl.ds(base, _ROWS)] = z
        return (s1 + jnp.sum(z, axis=1, keepdims=True),
                s2 + jnp.sum(z * z, axis=1, keepdims=True))

    zero = (jnp.zeros((o, 1), jnp.float32), jnp.zeros((o, 1), jnp.float32))
    s1, s2 = lax.fori_loop(0, nch, phase0, zero)

    def norm_consts(s1, s2):
        m = s1 * ninv
        v = s2 * ninv - m * m
        return m, lax.rsqrt(v + _EPS)

    m, sc = norm_consts(s1, s2)

    for w_ref, b_ref in ((w1_ref, b1_ref), (w2_ref, b2_ref)):
        def phase_mid(c, carry, w_ref=w_ref, b_ref=b_ref, m=m, sc=sc):
            s1, s2 = carry
            base = pl.multiple_of(c * _ROWS, _ROWS)
            z = z_ref[:, pl.ds(base, _ROWS)]
            x = jnp.maximum((z - m) * sc, 0.0)
            z2 = jnp.dot(w_ref[...], x, preferred_element_type=jnp.float32) + b_ref[...]
            z_ref[:, pl.ds(base, _ROWS)] = z2
            return (s1 + jnp.sum(z2, axis=1, keepdims=True),
                    s2 + jnp.sum(z2 * z2, axis=1, keepdims=True))

        s1, s2 = lax.fori_loop(0, nch, phase_mid, zero)
        m, sc = norm_consts(s1, s2)

    def phase_out(c, carry, m=m, sc=sc):
        base = pl.multiple_of(c * 2 * _ROWS, 2 * _ROWS)
        qbase = pl.multiple_of(c * 2 * rq, 2 * rq)
        z = z_ref[:, pl.ds(base, 2 * _ROWS)]
        x = jnp.maximum((z - m) * sc, 0.0)
        xr = x.T.reshape(2 * rq, _K, o)
        o_ref[0, :, pl.ds(qbase, 2 * rq)] = jnp.max(xr, axis=1).T
        return carry

    lax.fori_loop(0, nch // 2, phase_out, 0)


def _mlp(g, qexp, ws, bs):
    b, dch, rows = g.shape
    s = rows // _K
    o = ws[0].shape[0]

    def w_spec(w):
        return pl.BlockSpec(w.shape, lambda i: (0, 0))

    feat = pl.pallas_call(
        functools.partial(_mlp_body, rows, dch, o),
        grid=(b,),
        in_specs=[pl.BlockSpec((1, dch, rows), lambda i: (i, 0, 0)),
                  pl.BlockSpec((1, dch, rows), lambda i: (i, 0, 0)),
                  w_spec(ws[0]), w_spec(bs[0]),
                  w_spec(ws[1]), w_spec(bs[1]),
                  w_spec(ws[2]), w_spec(bs[2])],
        out_specs=pl.BlockSpec((1, o, s), lambda i: (i, 0, 0)),
        out_shape=jax.ShapeDtypeStruct((b, o, s), jnp.float32),
        scratch_shapes=[pltpu.VMEM((o, rows), jnp.float32)],
    )(g, qexp, ws[0], bs[0], ws[1], bs[1], ws[2], bs[2])
    return feat


def _pad_w(w, dch):
    o, c = w.shape
    return jnp.pad(w, ((0, 0), (0, dch - c)))


def _sa_layer(points3, feat_rows, npoint, ws_raw, bs_raw, dch):
    b, _, n = points3.shape
    c = feat_rows.shape[2]
    xyz_rows = jnp.transpose(points3, (0, 2, 1))
    table = jnp.concatenate(
        [xyz_rows, feat_rows, jnp.zeros((b, n, dch - 3 - c), jnp.float32)],
        axis=-1).reshape(b * n, dch)
    boff = (jnp.arange(b, dtype=jnp.int32) * n)
    fps_idx = _fps(jnp.transpose(points3, (1, 0, 2)), npoint)
    new_rows = _sc_gather(table, (fps_idx + boff[:, None]).reshape(-1))
    new_rows = new_rows.reshape(b, npoint, dch)
    new_xyz = new_rows[..., :3]
    idx = _knn(points3, new_xyz)
    grouped = _sc_gather(table, (idx + boff[:, None, None]).reshape(-1))
    grouped = jnp.transpose(grouped.reshape(b, npoint * _K, dch), (0, 2, 1))
    qpad = jnp.concatenate(
        [new_xyz, jnp.zeros((b, npoint, dch - 3), jnp.float32)], axis=-1)
    qexp = jnp.repeat(jnp.transpose(qpad, (0, 2, 1)), _K, axis=2)
    ws = [_pad_w(w, dch if i == 0 else ws_raw[i - 1].shape[0])
          for i, w in enumerate(ws_raw)]
    bs = [bb.reshape(-1, 1) for bb in bs_raw]
    feat = _mlp(grouped, qexp, ws, bs)
    return new_xyz, feat, fps_idx


def kernel(pc, feature,
           sa1_w0, sa1_b0, sa1_w1, sa1_b1, sa1_w2, sa1_b2,
           sa2_w0, sa2_b0, sa2_w1, sa2_b1, sa2_w2, sa2_b2):
    b, _, n = pc.shape
    feat_rows0 = jnp.transpose(feature, (0, 2, 1))
    new_xyz1, feat1, fps_idx1 = _sa_layer(
        pc, feat_rows0, n // 2, (sa1_w0, sa1_w1, sa1_w2),
        (sa1_b0, sa1_b1, sa1_b2), 16)
    pc_l1 = jnp.transpose(new_xyz1, (0, 2, 1))
    new_xyz2, feat2, fps_idx2 = _sa_layer(
        pc_l1, jnp.transpose(feat1, (0, 2, 1)), n // 4,
        (sa2_w0, sa2_w1, sa2_w2), (sa2_b0, sa2_b1, sa2_b2), 48)
    pc_l2 = jnp.transpose(new_xyz2, (0, 2, 1))
    return (pc, pc_l1, pc_l2, feat2, fps_idx1, fps_idx2)


# speedup vs baseline: 1.0317x; 1.0317x over previous
import functools

import jax
import jax.numpy as jnp
from jax import lax
from jax.experimental import pallas as pl
from jax.experimental.pallas import tpu as pltpu
from jax.experimental.pallas import tpu_sc as plsc

_K = 32
_EPS = 1e-5
_ROWS = 2048


def _fps_body(npoint, b, n, xyz_ref, iota_ref, idx_ref):
    x = xyz_ref[0]
    y = xyz_ref[1]
    z = xyz_ref[2]
    iota = iota_ref[...]
    eye = (lax.broadcasted_iota(jnp.int32, (b, b), 0)
           == lax.broadcasted_iota(jnp.int32, (b, b), 1))

    def _red(a, op):
        return op(op(a, axis=2, keepdims=True), axis=1, keepdims=True)

    def body(i, carry):
        dist, far = carry
        far_row = jnp.sum(jnp.where(eye, jnp.broadcast_to(far[:, :, 0], (b, b)), 0),
                          axis=0, keepdims=True)
        idx_ref[pl.ds(i, 1), :] = far_row
        oh = iota == far
        cx = _red(jnp.where(oh, x, 0.0), jnp.sum)
        cy = _red(jnp.where(oh, y, 0.0), jnp.sum)
        cz = _red(jnp.where(oh, z, 0.0), jnp.sum)
        d = (x - cx) ** 2 + (y - cy) ** 2 + (z - cz) ** 2
        dist = jnp.minimum(dist, d)
        m = _red(dist, jnp.max)
        far = _red(jnp.where(dist == m, iota, n), jnp.min)
        return dist, far

    lax.fori_loop(0, npoint, body,
                  (jnp.full((b, 8, n // 8), 1e10, jnp.float32),
                   jnp.zeros((b, 1, 1), jnp.int32)))


def _fps(xyz3, npoint):
    _, b, n = xyz3.shape
    xyz4 = xyz3.reshape(3, b, 8, n // 8)
    iota = jnp.arange(n, dtype=jnp.int32).reshape(8, n // 8)
    out = pl.pallas_call(
        functools.partial(_fps_body, npoint, b, n),
        out_shape=jax.ShapeDtypeStruct((npoint, b), jnp.int32),
    )(xyz4, iota)
    return out.T


def _knn_body(n, qb, x_ref, q_ref, o_ref):
    px = x_ref[0]
    sq = jnp.sum(px * px, axis=0, keepdims=True)
    q = q_ref[0]
    d = sq - 2.0 * jnp.dot(q, px, preferred_element_type=jnp.float32)
    iota = lax.broadcasted_iota(jnp.int32, (qb, n), 1)

    sels = []
    for _ in range(_K):
        m = jnp.min(d, axis=1, keepdims=True)
        sel = jnp.min(jnp.where(d == m, iota, n), axis=1, keepdims=True)
        sels.append(sel)
        d = jnp.where(iota == sel, jnp.inf, d)
    o_ref[0] = jnp.concatenate(sels, axis=1)


def _knn(points3, queries):
    b, _, n = points3.shape
    s = queries.shape[1]
    qb = 256
    return pl.pallas_call(
        functools.partial(_knn_body, n, qb),
        grid=(b, s // qb),
        in_specs=[
            pl.BlockSpec((1, 3, n), lambda i, j: (i, 0, 0)),
            pl.BlockSpec((1, qb, 3), lambda i, j: (i, j, 0)),
        ],
        out_specs=pl.BlockSpec((1, qb, _K), lambda i, j: (i, j, 0)),
        out_shape=jax.ShapeDtypeStruct((b, s, _K), jnp.int32),
    )(points3, queries)


def _sc_gather(table, idx):
    v, d = table.shape
    bt = idx.shape[0]
    info = plsc.get_sparse_core_info()
    nw = info.num_cores * info.num_subcores
    b_per_w = bt // nw
    cb = b_per_w
    while cb * d * 4 > 262144:
        cb //= 2
    nchunks = b_per_w // cb
    mesh = plsc.VectorSubcoreMesh(core_axis_name="c", subcore_axis_name="s")

    @functools.partial(
        pl.kernel, mesh=mesh,
        out_type=jax.ShapeDtypeStruct((bt, d), jnp.float32),
        compiler_params=pltpu.CompilerParams(use_tc_tiling_on_sc=False),
        scratch_types=[
            pltpu.VMEM((cb,), jnp.int32),
            pltpu.VMEM((cb, d), jnp.float32),
            pltpu.SemaphoreType.DMA,
        ],
    )
    def k(table_hbm, idx_hbm, out_hbm, idx_v, rows_v, sem):
        wid = lax.axis_index("s") * info.num_cores + lax.axis_index("c")
        base = wid * b_per_w

        def body(c, carry):
            off = base + c * cb
            pltpu.sync_copy(idx_hbm.at[pl.ds(off, cb)], idx_v)
            pltpu.async_copy(table_hbm.at[idx_v], rows_v, sem).wait()
            pltpu.sync_copy(rows_v, out_hbm.at[pl.ds(off, cb)])
            return carry

        lax.fori_loop(0, nchunks, body, 0)

    return k(table, idx)


def _mlp_body(rows, dch, o, g_ref, q_ref, w0_ref, b0_ref, w1_ref, b1_ref,
              w2_ref, b2_ref, o_ref, z_ref):
    nch = rows // _ROWS
    rq = _ROWS // _K
    ninv = 1.0 / rows

    def phase0(c, carry):
        s1, s2 = carry
        base = pl.multiple_of(c * _ROWS, _ROWS)
        x0 = g_ref[0, :, pl.ds(base, _ROWS)] - q_ref[0, :, pl.ds(base, _ROWS)]
        z = jnp.dot(w0_ref[...], x0, preferred_element_type=jnp.float32) + b0_ref[...]
        z_ref[:, pl.ds(base, _ROWS)] = z
        return (s1 + jnp.sum(z, axis=1, keepdims=True),
                s2 + jnp.sum(z * z, axis=1, keepdims=True))

    zero = (jnp.zeros((o, 1), jnp.float32), jnp.zeros((o, 1), jnp.float32))
    s1, s2 = lax.fori_loop(0, nch, phase0, zero)

    def norm_consts(s1, s2):
        m = s1 * ninv
        v = s2 * ninv - m * m
        return m, lax.rsqrt(v + _EPS)

    m, sc = norm_consts(s1, s2)

    for w_ref, b_ref in ((w1_ref, b1_ref), (w2_ref, b2_ref)):
        def phase_mid(c, carry, w_ref=w_ref, b_ref=b_ref, m=m, sc=sc):
            s1, s2 = carry
            base = pl.multiple_of(c * _ROWS, _ROWS)
            z = z_ref[:, pl.ds(base, _ROWS)]
            x = jnp.maximum((z - m) * sc, 0.0)
            z2 = jnp.dot(w_ref[...], x, preferred_element_type=jnp.float32) + b_ref[...]
            z_ref[:, pl.ds(base, _ROWS)] = z2
            return (s1 + jnp.sum(z2, axis=1, keepdims=True),
                    s2 + jnp.sum(z2 * z2, axis=1, keepdims=True))

        s1, s2 = lax.fori_loop(0, nch, phase_mid, zero)
        m, sc = norm_consts(s1, s2)

    def phase_out(c, carry, m=m, sc=sc):
        base = pl.multiple_of(c * 2 * _ROWS, 2 * _ROWS)
        qbase = pl.multiple_of(c * 2 * rq, 2 * rq)
        z = z_ref[:, pl.ds(base, 2 * _ROWS)]
        x = jnp.maximum((z - m) * sc, 0.0)
        xr = x.T.reshape(2 * rq, _K, o)
        o_ref[0, :, pl.ds(qbase, 2 * rq)] = jnp.max(xr, axis=1).T
        return carry

    lax.fori_loop(0, nch // 2, phase_out, 0)


def _mlp(g, qexp, ws, bs):
    b, dch, rows = g.shape
    s = rows // _K
    o = ws[0].shape[0]

    def w_spec(w):
        return pl.BlockSpec(w.shape, lambda i: (0, 0))

    feat = pl.pallas_call(
        functools.partial(_mlp_body, rows, dch, o),
        grid=(b,),
        in_specs=[pl.BlockSpec((1, dch, rows), lambda i: (i, 0, 0)),
                  pl.BlockSpec((1, dch, rows), lambda i: (i, 0, 0)),
                  w_spec(ws[0]), w_spec(bs[0]),
                  w_spec(ws[1]), w_spec(bs[1]),
                  w_spec(ws[2]), w_spec(bs[2])],
        out_specs=pl.BlockSpec((1, o, s), lambda i: (i, 0, 0)),
        out_shape=jax.ShapeDtypeStruct((b, o, s), jnp.float32),
        scratch_shapes=[pltpu.VMEM((o, rows), jnp.float32)],
    )(g, qexp, ws[0], bs[0], ws[1], bs[1], ws[2], bs[2])
    return feat


def _pad_w(w, dch):
    o, c = w.shape
    return jnp.pad(w, ((0, 0), (0, dch - c)))


def _sa_layer(points3, feat_rows, npoint, ws_raw, bs_raw, dch):
    b, _, n = points3.shape
    c = feat_rows.shape[2]
    xyz_rows = jnp.transpose(points3, (0, 2, 1))
    table = jnp.concatenate(
        [xyz_rows, feat_rows, jnp.zeros((b, n, dch - 3 - c), jnp.float32)],
        axis=-1).reshape(b * n, dch)
    boff = (jnp.arange(b, dtype=jnp.int32) * n)
    fps_idx = _fps(jnp.transpose(points3, (1, 0, 2)), npoint)
    new_rows = _sc_gather(table, (fps_idx + boff[:, None]).reshape(-1))
    new_rows = new_rows.reshape(b, npoint, dch)
    new_xyz = new_rows[..., :3]
    idx = _knn(points3, new_xyz)
    grouped = _sc_gather(table, (idx + boff[:, None, None]).reshape(-1))
    grouped = jnp.transpose(grouped.reshape(b, npoint * _K, dch), (0, 2, 1))
    qpad = jnp.concatenate(
        [new_xyz, jnp.zeros((b, npoint, dch - 3), jnp.float32)], axis=-1)
    qexp = jnp.repeat(jnp.transpose(qpad, (0, 2, 1)), _K, axis=2)
    ws = [_pad_w(w, dch if i == 0 else ws_raw[i - 1].shape[0])
          for i, w in enumerate(ws_raw)]
    bs = [bb.reshape(-1, 1) for bb in bs_raw]
    feat = _mlp(grouped, qexp, ws, bs)
    return new_xyz, feat, fps_idx


def kernel(pc, feature,
           sa1_w0, sa1_b0, sa1_w1, sa1_b1, sa1_w2, sa1_b2,
           sa2_w0, sa2_b0, sa2_w1, sa2_b1, sa2_w2, sa2_b2):
    b, _, n = pc.shape
    feat_rows0 = jnp.transpose(feature, (0, 2, 1))
    new_xyz1, feat1, fps_idx1 = _sa_layer(
        pc, feat_rows0, n // 2, (sa1_w0, sa1_w1, sa1_w2),
        (sa1_b0, sa1_b1, sa1_b2), 16)
    pc_l1 = jnp.transpose(new_xyz1, (0, 2, 1))
    new_xyz2, feat2, fps_idx2 = _sa_layer(
        pc_l1, jnp.transpose(feat1, (0, 2, 1)), n // 4,
        (sa2_w0, sa2_w1, sa2_w2), (sa2_b0, sa2_b1, sa2_b2), 48)
    pc_l2 = jnp.transpose(new_xyz2, (0, 2, 1))
    return (pc, pc_l1, pc_l2, feat2, fps_idx1, fps_idx2)
